# baseline (speedup 1.0000x reference)
import jax
import jax.numpy as jnp
from jax import lax
from jax.experimental import pallas as pl
from jax.experimental.pallas import tpu as pltpu
from jax.experimental.pallas import tpu_sc as plsc

NUM_ROWS = 16384
SEQ = 200
VOCAB = 100000
DIM = 48

NC, NS = 2, 16
NW = NC * NS
LANES = 16
TILE = 128
TPW = NUM_ROWS // NW
LT = TPW // TILE
KB = DIM // 8
NBUF = 2


def _emb_kernel(idx_hbm, table_hbm, out_hbm, idx_v, rows_v, tiles_v,
                gsem, ssem):
  wid = lax.axis_index("s") * NC + lax.axis_index("c")
  iota = lax.iota(jnp.int32, LANES)
  row_sel = [iota + c * LANES for c in range(TILE // LANES)]

  def load_and_fire(j, b):
    pltpu.sync_copy(idx_hbm.at[j, pl.ds(LT * wid, LT)], idx_v.at[b])
    for l in range(LT):
      pltpu.async_copy(
          table_hbm.at[idx_v.at[b, l]], rows_v.at[b, l], gsem.at[b])

  def drain_gathers(b):
    for _ in range(LT):
      pltpu.make_async_copy(
          table_hbm.at[pl.ds(0, TILE)], rows_v.at[b, 0], gsem.at[b]
      ).wait()

  def transpose(b):
    @plsc.parallel_loop(0, DIM, unroll=4)
    def kbody(k):
      kdiv = lax.div(k, 8)
      kmod = lax.rem(k, 8)
      colv = lax.broadcast_in_dim(k, (LANES,), ())
      for l in range(LT):
        for c in range(TILE // LANES):
          v = plsc.load_gather(rows_v.at[b, l], [row_sel[c], colv])
          tiles_v[b, l, kdiv, kmod, pl.ds(c * LANES, LANES)] = v

  def stores(j, b):
    for l in range(LT):
      pltpu.async_copy(
          tiles_v.at[b, l], out_hbm.at[j, :, LT * wid + l], ssem.at[b])

  def drain_stores(b):
    for _ in range(LT):
      pltpu.make_async_copy(
          tiles_v.at[b, 0], out_hbm.at[0, :, 0], ssem.at[b]).wait()

  load_and_fire(0, 0)

  def body(g, carry):
    b = lax.rem(g, NBUF)
    nb = lax.rem(g + 1, NBUF)

    @pl.when(g >= NBUF)
    def _free_tiles_buffer():
      drain_stores(b)

    @pl.when(g + 1 < SEQ)
    def _prefetch():
      load_and_fire(g + 1, nb)

    drain_gathers(b)
    transpose(b)
    stores(g, b)
    return carry

  lax.fori_loop(0, SEQ, body, 0)
  drain_stores(0)
  drain_stores(1)


@jax.jit
def _emb(idx3, weight):
  mesh = plsc.VectorSubcoreMesh(core_axis_name="c", subcore_axis_name="s")
  kfn = pl.kernel(
      _emb_kernel,
      out_type=jax.ShapeDtypeStruct((SEQ, KB, NUM_ROWS // TILE, 8, TILE),
                                    jnp.float32),
      mesh=mesh,
      scratch_types=[
          pltpu.VMEM((NBUF, LT, TILE), jnp.int32),
          pltpu.VMEM((NBUF, LT, TILE, DIM), jnp.float32),
          pltpu.VMEM((NBUF, LT, KB, 8, TILE), jnp.float32),
          pltpu.SemaphoreType.DMA((NBUF,)),
          pltpu.SemaphoreType.DMA((NBUF,)),
      ],
      compiler_params=pltpu.CompilerParams(
          use_tc_tiling_on_sc=False, needs_layout_passes=False),
  )
  return kfn(idx3, weight)


def kernel(idx, weight):
  idx3 = jnp.transpose(idx).reshape(SEQ, NUM_ROWS // TILE, TILE)
  p = _emb(idx3, weight)
  return jnp.transpose(p, (2, 4, 0, 1, 3)).reshape(NUM_ROWS, SEQ, DIM)


# speedup vs baseline: 1.2961x; 1.2961x over previous
import jax
import jax.numpy as jnp
from jax import lax
from jax.experimental import pallas as pl
from jax.experimental.pallas import tpu as pltpu
from jax.experimental.pallas import tpu_sc as plsc

NUM_ROWS = 16384
SEQ = 200
VOCAB = 100000
DIM = 48

NC, NS = 2, 16
NW = NC * NS
LANES = 16
TILE = 128
TPW = NUM_ROWS // NW
LT = TPW // TILE
KB = DIM // 8
NBUF = 2


PITCH = 49


def _emb_kernel(idx_hbm, table_hbm, out_hbm, idx_v, rows_v, tiles_v, skw_v,
                gsem, ssem):
  wid = lax.axis_index("s") * NC + lax.axis_index("c")
  iota = lax.iota(jnp.int32, LANES)
  row_sel = [(iota + c * LANES) * PITCH for c in range(TILE // LANES)]

  def load_and_fire(j, b):
    pltpu.sync_copy(idx_hbm.at[j, pl.ds(LT * wid, LT)], idx_v.at[b])
    for l in range(LT):
      pltpu.async_copy(
          table_hbm.at[idx_v.at[b, l]], rows_v.at[b, l], gsem.at[b])

  def drain_gathers(b):
    for _ in range(LT):
      pltpu.make_async_copy(
          table_hbm.at[pl.ds(0, TILE)], rows_v.at[b, 0], gsem.at[b]
      ).wait()

  def transpose(b):
    @plsc.parallel_loop(0, TILE)
    def sbody(ir):
      for l in range(LT):
        for m in range(DIM // LANES):
          skw_v[l, pl.ds(ir * PITCH + m * LANES, LANES)] = (
              rows_v[b, l, ir, pl.ds(m * LANES, LANES)])

    @plsc.parallel_loop(0, DIM)
    def kbody(k):
      kdiv = lax.div(k, 8)
      kmod = lax.rem(k, 8)
      colv = lax.broadcast_in_dim(k, (LANES,), ())
      for l in range(LT):
        for c in range(TILE // LANES):
          v = plsc.load_gather(skw_v.at[l], [row_sel[c] + colv])
          tiles_v[b, l, kdiv, kmod, pl.ds(c * LANES, LANES)] = v

  def stores(j, b):
    for l in range(LT):
      pltpu.async_copy(
          tiles_v.at[b, l], out_hbm.at[j, :, LT * wid + l], ssem.at[b])

  def drain_stores(b):
    for _ in range(LT):
      pltpu.make_async_copy(
          tiles_v.at[b, 0], out_hbm.at[0, :, 0], ssem.at[b]).wait()

  load_and_fire(0, 0)

  def body(g, carry):
    b = lax.rem(g, NBUF)
    nb = lax.rem(g + 1, NBUF)

    @pl.when(g >= NBUF)
    def _free_tiles_buffer():
      drain_stores(b)

    @pl.when(g + 1 < SEQ)
    def _prefetch():
      load_and_fire(g + 1, nb)

    drain_gathers(b)
    transpose(b)
    stores(g, b)
    return carry

  lax.fori_loop(0, SEQ, body, 0)
  drain_stores(0)
  drain_stores(1)


@jax.jit
def _emb(idx3, weight):
  mesh = plsc.VectorSubcoreMesh(core_axis_name="c", subcore_axis_name="s")
  kfn = pl.kernel(
      _emb_kernel,
      out_type=jax.ShapeDtypeStruct((SEQ, KB, NUM_ROWS // TILE, 8, TILE),
                                    jnp.float32),
      mesh=mesh,
      scratch_types=[
          pltpu.VMEM((NBUF, LT, TILE), jnp.int32),
          pltpu.VMEM((NBUF, LT, TILE, DIM), jnp.float32),
          pltpu.VMEM((NBUF, LT, KB, 8, TILE), jnp.float32),
          pltpu.VMEM((LT, TILE * PITCH), jnp.float32),
          pltpu.SemaphoreType.DMA((NBUF,)),
          pltpu.SemaphoreType.DMA((NBUF,)),
      ],
      compiler_params=pltpu.CompilerParams(
          use_tc_tiling_on_sc=False, needs_layout_passes=False),
  )
  return kfn(idx3, weight)


def kernel(idx, weight):
  idx3 = jnp.transpose(idx).reshape(SEQ, NUM_ROWS // TILE, TILE)
  p = _emb(idx3, weight)
  return jnp.transpose(p, (2, 4, 0, 1, 3)).reshape(NUM_ROWS, SEQ, DIM)


# speedup vs baseline: 1.5060x; 1.1619x over previous
import jax
import jax.numpy as jnp
from jax import lax
from jax.experimental import pallas as pl
from jax.experimental.pallas import tpu as pltpu
from jax.experimental.pallas import tpu_sc as plsc

NUM_ROWS = 16384
SEQ = 200
VOCAB = 100000
DIM = 48

NC, NS = 2, 16
NW = NC * NS
LANES = 16
TILE = 128
TPW = NUM_ROWS // NW
LT = TPW // TILE
KB = DIM // 8
NBUF = 2


PITCH = 49


def _emb_kernel(idx_hbm, table_hbm, out_hbm, idx_v, rows_v, tiles_v, skw_v,
                gsem, ssem, isem):
  wid = lax.axis_index("s") * NC + lax.axis_index("c")
  iota = lax.iota(jnp.int32, LANES)
  row_sel = [(iota + c * LANES) * PITCH for c in range(TILE // LANES)]

  def load_idx(j, b):
    pltpu.async_copy(idx_hbm.at[j, pl.ds(LT * wid, LT)], idx_v.at[b], isem)

  def drain_idx(b):
    pltpu.make_async_copy(
        idx_hbm.at[0, pl.ds(0, LT)], idx_v.at[b], isem).wait()

  def fire_gathers(b):
    for l in range(LT):
      pltpu.async_copy(
          table_hbm.at[idx_v.at[b, l]], rows_v.at[b, l], gsem.at[b])

  def drain_gathers(b):
    for _ in range(LT):
      pltpu.make_async_copy(
          table_hbm.at[pl.ds(0, TILE)], rows_v.at[b, 0], gsem.at[b]
      ).wait()

  def transpose(b):
    @plsc.parallel_loop(0, TILE)
    def sbody(ir):
      for l in range(LT):
        for m in range(DIM // LANES):
          skw_v[l, pl.ds(ir * PITCH + m * LANES, LANES)] = (
              rows_v[b, l, ir, pl.ds(m * LANES, LANES)])

    @plsc.parallel_loop(0, DIM)
    def kbody(k):
      kdiv = lax.div(k, 8)
      kmod = lax.rem(k, 8)
      colv = lax.broadcast_in_dim(k, (LANES,), ())
      for l in range(LT):
        for c in range(TILE // LANES):
          v = plsc.load_gather(skw_v.at[l], [row_sel[c] + colv])
          tiles_v[b, l, kdiv, kmod, pl.ds(c * LANES, LANES)] = v

  def stores(j, b):
    for l in range(LT):
      pltpu.async_copy(
          tiles_v.at[b, l], out_hbm.at[j, :, LT * wid + l], ssem.at[b])

  def drain_stores(b):
    for _ in range(LT):
      pltpu.make_async_copy(
          tiles_v.at[b, 0], out_hbm.at[0, :, 0], ssem.at[b]).wait()

  load_idx(0, 0)
  drain_idx(0)
  fire_gathers(0)
  load_idx(1, 1)

  def body(g, carry):
    b = lax.rem(g, NBUF)
    nb = lax.rem(g + 1, NBUF)

    @pl.when(g >= NBUF)
    def _free_tiles_buffer():
      drain_stores(b)

    @pl.when(g + 1 < SEQ)
    def _prefetch():
      drain_idx(nb)
      fire_gathers(nb)

    drain_gathers(b)

    @pl.when(g + 2 < SEQ)
    def _prefetch_idx():
      load_idx(g + 2, b)

    transpose(b)
    stores(g, b)
    return carry

  lax.fori_loop(0, SEQ, body, 0)
  drain_stores(0)
  drain_stores(1)


@jax.jit
def _emb(idx3, weight):
  mesh = plsc.VectorSubcoreMesh(core_axis_name="c", subcore_axis_name="s")
  kfn = pl.kernel(
      _emb_kernel,
      out_type=jax.ShapeDtypeStruct((SEQ, KB, NUM_ROWS // TILE, 8, TILE),
                                    jnp.float32),
      mesh=mesh,
      scratch_types=[
          pltpu.VMEM((NBUF, LT, TILE), jnp.int32),
          pltpu.VMEM((NBUF, LT, TILE, DIM), jnp.float32),
          pltpu.VMEM((NBUF, LT, KB, 8, TILE), jnp.float32),
          pltpu.VMEM((LT, TILE * PITCH), jnp.float32),
          pltpu.SemaphoreType.DMA((NBUF,)),
          pltpu.SemaphoreType.DMA((NBUF,)),
          pltpu.SemaphoreType.DMA,
      ],
      compiler_params=pltpu.CompilerParams(
          use_tc_tiling_on_sc=False, needs_layout_passes=False),
  )
  return kfn(idx3, weight)


def kernel(idx, weight):
  idx3 = jnp.transpose(idx).reshape(SEQ, NUM_ROWS // TILE, TILE)
  p = _emb(idx3, weight)
  return jnp.transpose(p, (2, 4, 0, 1, 3)).reshape(NUM_ROWS, SEQ, DIM)


# speedup vs baseline: 1.5145x; 1.0057x over previous
import jax
import jax.numpy as jnp
from jax import lax
from jax.experimental import pallas as pl
from jax.experimental.pallas import tpu as pltpu
from jax.experimental.pallas import tpu_sc as plsc

NUM_ROWS = 16384
SEQ = 200
VOCAB = 100000
DIM = 48

NC, NS = 2, 16
NW = NC * NS
LANES = 16
TILE = 128
TPW = NUM_ROWS // NW
LT = TPW // TILE
KB = DIM // 8
NBUF = 2


PITCH = 49


def _emb_kernel(idx_hbm, table_hbm, out_hbm, idx_v, rows_v, tiles_v, skw_v,
                gsem, ssem, isem):
  wid = lax.axis_index("s") * NC + lax.axis_index("c")
  iota = lax.iota(jnp.int32, LANES)
  row_sel = [(iota + c * LANES) * PITCH for c in range(TILE // LANES)]

  def load_idx(j, b):
    pltpu.async_copy(
        idx_hbm.at[lax.div(j, 8), pl.ds(LT * wid, LT), lax.rem(j, 8)],
        idx_v.at[b], isem)

  def drain_idx(b):
    pltpu.make_async_copy(
        idx_hbm.at[0, pl.ds(0, LT), 0], idx_v.at[b], isem).wait()

  def fire_gathers(b):
    for l in range(LT):
      pltpu.async_copy(
          table_hbm.at[idx_v.at[b, l]], rows_v.at[b, l], gsem.at[b])

  def drain_gathers(b):
    for _ in range(LT):
      pltpu.make_async_copy(
          table_hbm.at[pl.ds(0, TILE)], rows_v.at[b, 0], gsem.at[b]
      ).wait()

  def transpose(b):
    @plsc.parallel_loop(0, TILE)
    def sbody(ir):
      for l in range(LT):
        for m in range(DIM // LANES):
          skw_v[l, pl.ds(ir * PITCH + m * LANES, LANES)] = (
              rows_v[b, l, ir, pl.ds(m * LANES, LANES)])

    @plsc.parallel_loop(0, DIM)
    def kbody(k):
      kdiv = lax.div(k, 8)
      kmod = lax.rem(k, 8)
      colv = lax.broadcast_in_dim(k, (LANES,), ())
      for l in range(LT):
        for c in range(TILE // LANES):
          v = plsc.load_gather(skw_v.at[l], [row_sel[c] + colv])
          tiles_v[b, l, kdiv, kmod, pl.ds(c * LANES, LANES)] = v

  def stores(j, b):
    for l in range(LT):
      pltpu.async_copy(
          tiles_v.at[b, l], out_hbm.at[j, :, LT * wid + l], ssem.at[b])

  def drain_stores(b):
    for _ in range(LT):
      pltpu.make_async_copy(
          tiles_v.at[b, 0], out_hbm.at[0, :, 0], ssem.at[b]).wait()

  load_idx(0, 0)
  drain_idx(0)
  fire_gathers(0)
  load_idx(1, 1)

  def body(g, carry):
    b = lax.rem(g, NBUF)
    nb = lax.rem(g + 1, NBUF)

    @pl.when(g >= NBUF)
    def _free_tiles_buffer():
      drain_stores(b)

    @pl.when(g + 1 < SEQ)
    def _prefetch():
      drain_idx(nb)
      fire_gathers(nb)

    drain_gathers(b)

    @pl.when(g + 2 < SEQ)
    def _prefetch_idx():
      load_idx(g + 2, b)

    transpose(b)
    stores(g, b)
    return carry

  lax.fori_loop(0, SEQ, body, 0)
  drain_stores(0)
  drain_stores(1)


@jax.jit
def _emb(idxP, weight):
  mesh = plsc.VectorSubcoreMesh(core_axis_name="c", subcore_axis_name="s")
  kfn = pl.kernel(
      _emb_kernel,
      out_type=jax.ShapeDtypeStruct((SEQ, KB, NUM_ROWS // TILE, 8, TILE),
                                    jnp.float32),
      mesh=mesh,
      scratch_types=[
          pltpu.VMEM((NBUF, LT, TILE), jnp.int32),
          pltpu.VMEM((NBUF, LT, TILE, DIM), jnp.float32),
          pltpu.VMEM((NBUF, LT, KB, 8, TILE), jnp.float32),
          pltpu.VMEM((LT, TILE * PITCH), jnp.float32),
          pltpu.SemaphoreType.DMA((NBUF,)),
          pltpu.SemaphoreType.DMA((NBUF,)),
          pltpu.SemaphoreType.DMA,
      ],
      compiler_params=pltpu.CompilerParams(
          use_tc_tiling_on_sc=False, needs_layout_passes=False),
  )
  return kfn(idxP, weight)


def kernel(idx, weight):
  idxP = jnp.transpose(
      jnp.transpose(idx).reshape(SEQ // 8, 8, NUM_ROWS // TILE, TILE),
      (0, 2, 1, 3))
  p = _emb(idxP, weight)
  return jnp.transpose(p, (2, 4, 0, 1, 3)).reshape(NUM_ROWS, SEQ, DIM)


# speedup vs baseline: 1.5161x; 1.0011x over previous
import jax
import jax.numpy as jnp
from jax import lax
from jax.experimental import pallas as pl
from jax.experimental.pallas import tpu as pltpu
from jax.experimental.pallas import tpu_sc as plsc

NUM_ROWS = 16384
SEQ = 200
VOCAB = 100000
DIM = 48

NC, NS = 2, 16
NW = NC * NS
LANES = 16
TILE = 128
TPW = NUM_ROWS // NW
LT = TPW // TILE
KB = DIM // 8
NBUF = 2


PITCH = 49


def _emb_kernel(idx_hbm, table_hbm, out_hbm, idx_v, rows_v, tiles_v, skw_v,
                gsem, ssem, isem):
  wid = lax.axis_index("s") * NC + lax.axis_index("c")
  iota = lax.iota(jnp.int32, LANES)
  row_sel = [(iota + c * LANES) * PITCH for c in range(TILE // LANES)]

  def load_idx(j, b):
    pltpu.async_copy(
        idx_hbm.at[lax.div(j, 8), pl.ds(LT * wid, LT), lax.rem(j, 8)],
        idx_v.at[b], isem)

  def drain_idx(b):
    pltpu.make_async_copy(
        idx_hbm.at[0, pl.ds(0, LT), 0], idx_v.at[b], isem).wait()

  def fire_gathers(b):
    for l in range(LT):
      pltpu.async_copy(
          table_hbm.at[idx_v.at[b, l]], rows_v.at[b, l], gsem.at[b])

  def drain_gathers(b):
    for _ in range(LT):
      pltpu.make_async_copy(
          table_hbm.at[pl.ds(0, TILE)], rows_v.at[b, 0], gsem.at[b]
      ).wait()

  def transpose(b):
    @plsc.parallel_loop(0, TILE, unroll=2)
    def sbody(ir):
      for l in range(LT):
        for m in range(DIM // LANES):
          skw_v[l, pl.ds(ir * PITCH + m * LANES, LANES)] = (
              rows_v[b, l, ir, pl.ds(m * LANES, LANES)])

    @plsc.parallel_loop(0, DIM)
    def kbody(k):
      kdiv = lax.div(k, 8)
      kmod = lax.rem(k, 8)
      colv = lax.broadcast_in_dim(k, (LANES,), ())
      for l in range(LT):
        for c in range(TILE // LANES):
          v = plsc.load_gather(skw_v.at[l], [row_sel[c] + colv])
          tiles_v[b, l, kdiv, kmod, pl.ds(c * LANES, LANES)] = v

  def stores(j, b):
    for l in range(LT):
      pltpu.async_copy(
          tiles_v.at[b, l], out_hbm.at[j, :, LT * wid + l], ssem.at[b])

  def drain_stores(b):
    for _ in range(LT):
      pltpu.make_async_copy(
          tiles_v.at[b, 0], out_hbm.at[0, :, 0], ssem.at[b]).wait()

  load_idx(0, 0)
  drain_idx(0)
  fire_gathers(0)
  load_idx(1, 1)

  def body(g, carry):
    b = lax.rem(g, NBUF)
    nb = lax.rem(g + 1, NBUF)

    @pl.when(g >= NBUF)
    def _free_tiles_buffer():
      drain_stores(b)

    @pl.when(g + 1 < SEQ)
    def _prefetch():
      drain_idx(nb)
      fire_gathers(nb)

    drain_gathers(b)

    @pl.when(g + 2 < SEQ)
    def _prefetch_idx():
      load_idx(g + 2, b)

    transpose(b)
    stores(g, b)
    return carry

  lax.fori_loop(0, SEQ, body, 0)
  drain_stores(0)
  drain_stores(1)


@jax.jit
def _emb(idxP, weight):
  mesh = plsc.VectorSubcoreMesh(core_axis_name="c", subcore_axis_name="s")
  kfn = pl.kernel(
      _emb_kernel,
      out_type=jax.ShapeDtypeStruct((SEQ, KB, NUM_ROWS // TILE, 8, TILE),
                                    jnp.float32),
      mesh=mesh,
      scratch_types=[
          pltpu.VMEM((NBUF, LT, TILE), jnp.int32),
          pltpu.VMEM((NBUF, LT, TILE, DIM), jnp.float32),
          pltpu.VMEM((NBUF, LT, KB, 8, TILE), jnp.float32),
          pltpu.VMEM((LT, TILE * PITCH), jnp.float32),
          pltpu.SemaphoreType.DMA((NBUF,)),
          pltpu.SemaphoreType.DMA((NBUF,)),
          pltpu.SemaphoreType.DMA,
      ],
      compiler_params=pltpu.CompilerParams(
          use_tc_tiling_on_sc=False, needs_layout_passes=False),
  )
  return kfn(idxP, weight)


def kernel(idx, weight):
  idxP = jnp.transpose(
      jnp.transpose(idx).reshape(SEQ // 8, 8, NUM_ROWS // TILE, TILE),
      (0, 2, 1, 3))
  p = _emb(idxP, weight)
  return jnp.transpose(p, (2, 4, 0, 1, 3)).reshape(NUM_ROWS, SEQ, DIM)
